# baseline (speedup 1.0000x reference)
import functools

import jax
import jax.numpy as jnp
from jax import lax
from jax.experimental import pallas as pl
from jax.experimental.pallas import tpu as pltpu
from jax.experimental.pallas import tpu_sc as plsc

W_SIZE = 5
STRIDE = 4
PADDING = 2
RIGHT_EXTRA = 1

_NW = 32
_CH = 128
_LANES = 16
_NB = 4
_RB = 16


def _transpose_to_rows(x):
    N, C, H, W = x.shape
    br = _RB * W
    nblk = (N * H) // _RB
    hb = H // _RB

    def body(x_ref, o_ref):
        i = pl.program_id(0)

        @pl.when(i < nblk)
        def _():
            blk = x_ref[0].reshape(C, br)
            o_ref[...] = jnp.transpose(blk, (1, 0))

        @pl.when(i == nblk)
        def _():
            o_ref[...] = jnp.zeros((br, C), jnp.float32)

    in_spec = pl.BlockSpec(
        (1, C, _RB, W),
        lambda i: (jnp.where(i < nblk, i // hb, 0), 0,
                   jnp.where(i < nblk, i % hb, 0), 0))
    out_spec = pl.BlockSpec((br, C), lambda i: (i, 0))
    return pl.pallas_call(
        body,
        grid=(nblk + 1,),
        in_specs=[in_spec],
        out_specs=out_spec,
        out_shape=jax.ShapeDtypeStruct(((nblk + 1) * br, C), jnp.float32),
    )(x)


def _retile(flat, k):
    R, C = flat.shape
    M = R // k
    bm = 8
    assert M % bm == 0

    def body(x_ref, o_ref):
        o_ref[...] = x_ref[...].reshape(bm, k, C)

    return pl.pallas_call(
        body,
        grid=(M // bm,),
        in_specs=[pl.BlockSpec((bm * k, C), lambda i: (i, 0))],
        out_specs=pl.BlockSpec((bm, k, C), lambda i: (i, 0, 0)),
        out_shape=jax.ShapeDtypeStruct((M, k, C), jnp.float32),
    )(flat)


def _build_sc_gather(M, C, H, W, ow, w, p, zr):
    kx = w * w
    hw = H * W
    nfc, tail = divmod(M * kx, _CH)
    per = -(-nfc // _NW)
    mp8 = -(-M // 8) * 8

    assert ow & (ow - 1) == 0
    ow_shift = ow.bit_length() - 1
    oh = ow
    assert (oh - 1) * STRIDE + (w - 1) - p < H

    mesh = plsc.VectorSubcoreMesh(core_axis_name="c", subcore_axis_name="s")

    @functools.partial(
        pl.kernel,
        mesh=mesh,
        compiler_params=pltpu.CompilerParams(needs_layout_passes=False),
        out_type=jax.ShapeDtypeStruct((M * kx, C), jnp.float32),
        scratch_types=[
            pltpu.VMEM((mp8,), jnp.int32),
            pltpu.VMEM((mp8,), jnp.int32),
        ] + [pltpu.VMEM((_CH,), jnp.int32)] * _NB
          + [pltpu.VMEM((_CH, C), jnp.float32)] * _NB
          + [pltpu.SemaphoreType.DMA] * _NB,
    )
    def sc_fn(tab, b_hbm, p_hbm, out, b_v, pos_v, *bufs):
        idxb = bufs[:_NB]
        rows = bufs[_NB:2 * _NB]
        sems = bufs[2 * _NB:]
        wid = lax.axis_index("s") * 2 + lax.axis_index("c")
        lanes = lax.iota(jnp.int32, _LANES)
        zrv = jnp.full((_LANES,), zr, jnp.int32)
        mmax = jnp.full((_LANES,), M - 1, jnp.int32)

        pltpu.sync_copy(b_hbm, b_v)
        pltpu.sync_copy(p_hbm, pos_v)

        def fill_chunk(buf, c):
            kmax = jnp.full((_LANES,), kx - 1, jnp.int32)

            def grp(g, carry):
                r = c * _CH + g * _LANES + lanes
                m = jnp.minimum(r // kx, mmax)
                kk = jnp.minimum(r - m * kx, kmax)
                bv = plsc.load_gather(b_v, [m])
                pv = plsc.load_gather(pos_v, [m])
                oy = pv >> ow_shift
                ox = pv & (ow - 1)
                dy = kk // w
                dx = kk - dy * w
                y0 = oy * STRIDE + (dy - p)
                x0 = ox * STRIDE + (dx - p)
                valid = (y0 >= 0) & (x0 >= 0)
                row = jnp.where(valid, bv * hw + y0 * W + x0, zrv)
                buf[pl.ds(g * _LANES, _LANES)] = row
                return carry

            lax.fori_loop(0, _CH // _LANES, grp, 0)

        @pl.when(wid == 0)
        def _():
            fill_chunk(idxb[0], nfc)
            pltpu.async_copy(tab.at[idxb[0]], rows[0], sems[0]).wait()
            pltpu.sync_copy(rows[0].at[pl.ds(0, tail)],
                            out.at[pl.ds(nfc * _CH, tail)])

        start = (wid * (nfc - per)) // (_NW - 1)

        def wait64(b):
            pltpu.make_async_copy(out.at[pl.ds(0, _CH)],
                                  rows[b], sems[b]).wait()

        def fire_gather(b):
            pltpu.async_copy(tab.at[idxb[b]], rows[b], sems[b])

        def fire_write(c, b):
            off = pl.multiple_of((start + c) * _CH, _CH)
            pltpu.async_copy(rows[b], out.at[pl.ds(off, _CH)], sems[b])

        for b in range(_NB):
            fill_chunk(idxb[b], start + b)
            fire_gather(b)

        nq, _rem = divmod(per, _NB)

        def quad(q, carry):
            for b in range(_NB):
                c = q * _NB + b
                wait64(b)
                fire_write(c, b)

                @pl.when(c + _NB < per)
                def _():
                    fill_chunk(idxb[b], start + c + _NB)
                    wait64(b)
                    fire_gather(b)
            return carry

        lax.fori_loop(0, nq, quad, 0)
        for c in range(_NB * nq, per):
            b = c % _NB
            wait64(b)
            fire_write(c, b)
        for b in range(_NB):
            wait64(b)

    return mp8, sc_fn


def kernel(x0, x1, b_idxes, i_idxes, j_idxes):
    w0 = W_SIZE
    e = RIGHT_EXTRA
    w1 = w0 + 2 * e
    p0 = PADDING
    p1 = PADDING + e
    N, C, H, W = x0.shape
    ow = (W + 2 * p0 - w0) // STRIDE + 1
    M = b_idxes.shape[0]

    zr = N * H * W
    mp8, sc_fn0 = _build_sc_gather(M, C, H, W, ow, w0, p0, zr)
    _, sc_fn1 = _build_sc_gather(M, C, H, W, ow, w1, p1, zr)
    pad = mp8 - M
    b = jnp.pad(b_idxes.astype(jnp.int32), (0, pad))
    ii = jnp.pad(i_idxes.astype(jnp.int32), (0, pad))
    jj = jnp.pad(j_idxes.astype(jnp.int32), (0, pad))

    t1 = _transpose_to_rows(x1)
    out1f = sc_fn1(t1, b, jj)
    t0 = _transpose_to_rows(x0)
    out0f = sc_fn0(t0, b, ii)
    return (out0f.reshape(M, w0 * w0, C),
            out1f.reshape(M, w1 * w1, C))


# speedup vs baseline: 1.0107x; 1.0107x over previous
import functools

import jax
import jax.numpy as jnp
from jax import lax
from jax.experimental import pallas as pl
from jax.experimental.pallas import tpu as pltpu
from jax.experimental.pallas import tpu_sc as plsc

W_SIZE = 5
STRIDE = 4
PADDING = 2
RIGHT_EXTRA = 1

_NW = 32
_CH = 128
_LANES = 16
_NB = 4
_RB = 32


def _transpose_to_rows(x):
    N, C, H, W = x.shape
    br = _RB * W
    nblk = (N * H) // _RB
    hb = H // _RB

    def body(x_ref, o_ref):
        i = pl.program_id(0)

        @pl.when(i < nblk)
        def _():
            blk = x_ref[0].reshape(C, br)
            o_ref[...] = jnp.transpose(blk, (1, 0))

        @pl.when(i == nblk)
        def _():
            o_ref[...] = jnp.zeros((br, C), jnp.float32)

    in_spec = pl.BlockSpec(
        (1, C, _RB, W),
        lambda i: (jnp.where(i < nblk, i // hb, 0), 0,
                   jnp.where(i < nblk, i % hb, 0), 0))
    out_spec = pl.BlockSpec((br, C), lambda i: (i, 0))
    return pl.pallas_call(
        body,
        grid=(nblk + 1,),
        in_specs=[in_spec],
        out_specs=out_spec,
        out_shape=jax.ShapeDtypeStruct(((nblk + 1) * br, C), jnp.float32),
    )(x)


def _retile(flat, k):
    R, C = flat.shape
    M = R // k
    bm = 8
    assert M % bm == 0

    def body(x_ref, o_ref):
        o_ref[...] = x_ref[...].reshape(bm, k, C)

    return pl.pallas_call(
        body,
        grid=(M // bm,),
        in_specs=[pl.BlockSpec((bm * k, C), lambda i: (i, 0))],
        out_specs=pl.BlockSpec((bm, k, C), lambda i: (i, 0, 0)),
        out_shape=jax.ShapeDtypeStruct((M, k, C), jnp.float32),
    )(flat)


def _build_sc_gather(M, C, H, W, ow, w, p, zr):
    kx = w * w
    hw = H * W
    nfc, tail = divmod(M * kx, _CH)
    per = -(-nfc // _NW)
    mp8 = -(-M // 8) * 8

    assert ow & (ow - 1) == 0
    ow_shift = ow.bit_length() - 1
    oh = ow
    assert (oh - 1) * STRIDE + (w - 1) - p < H

    mesh = plsc.VectorSubcoreMesh(core_axis_name="c", subcore_axis_name="s")

    @functools.partial(
        pl.kernel,
        mesh=mesh,
        compiler_params=pltpu.CompilerParams(needs_layout_passes=False),
        out_type=jax.ShapeDtypeStruct((M * kx, C), jnp.float32),
        scratch_types=[
            pltpu.VMEM((mp8,), jnp.int32),
            pltpu.VMEM((mp8,), jnp.int32),
        ] + [pltpu.VMEM((_CH,), jnp.int32)] * _NB
          + [pltpu.VMEM((_CH, C), jnp.float32)] * _NB
          + [pltpu.SemaphoreType.DMA] * _NB,
    )
    def sc_fn(tab, b_hbm, p_hbm, out, b_v, pos_v, *bufs):
        idxb = bufs[:_NB]
        rows = bufs[_NB:2 * _NB]
        sems = bufs[2 * _NB:]
        wid = lax.axis_index("s") * 2 + lax.axis_index("c")
        lanes = lax.iota(jnp.int32, _LANES)
        zrv = jnp.full((_LANES,), zr, jnp.int32)
        mmax = jnp.full((_LANES,), M - 1, jnp.int32)

        pltpu.sync_copy(b_hbm, b_v)
        pltpu.sync_copy(p_hbm, pos_v)

        def fill_chunk(buf, c):
            kmax = jnp.full((_LANES,), kx - 1, jnp.int32)

            def grp(g, carry):
                r = c * _CH + g * _LANES + lanes
                m = jnp.minimum(r // kx, mmax)
                kk = jnp.minimum(r - m * kx, kmax)
                bv = plsc.load_gather(b_v, [m])
                pv = plsc.load_gather(pos_v, [m])
                oy = pv >> ow_shift
                ox = pv & (ow - 1)
                dy = kk // w
                dx = kk - dy * w
                y0 = oy * STRIDE + (dy - p)
                x0 = ox * STRIDE + (dx - p)
                valid = (y0 >= 0) & (x0 >= 0)
                row = jnp.where(valid, bv * hw + y0 * W + x0, zrv)
                buf[pl.ds(g * _LANES, _LANES)] = row
                return carry

            lax.fori_loop(0, _CH // _LANES, grp, 0)

        @pl.when(wid == 0)
        def _():
            fill_chunk(idxb[0], nfc)
            pltpu.async_copy(tab.at[idxb[0]], rows[0], sems[0]).wait()
            pltpu.sync_copy(rows[0].at[pl.ds(0, tail)],
                            out.at[pl.ds(nfc * _CH, tail)])

        start = (wid * (nfc - per)) // (_NW - 1)

        def wait64(b):
            pltpu.make_async_copy(out.at[pl.ds(0, _CH)],
                                  rows[b], sems[b]).wait()

        def fire_gather(b):
            pltpu.async_copy(tab.at[idxb[b]], rows[b], sems[b])

        def fire_write(c, b):
            off = pl.multiple_of((start + c) * _CH, _CH)
            pltpu.async_copy(rows[b], out.at[pl.ds(off, _CH)], sems[b])

        for b in range(_NB):
            fill_chunk(idxb[b], start + b)
            fire_gather(b)

        nq, _rem = divmod(per, _NB)

        def quad(q, carry):
            for b in range(_NB):
                c = q * _NB + b
                wait64(b)
                fire_write(c, b)

                @pl.when(c + _NB < per)
                def _():
                    fill_chunk(idxb[b], start + c + _NB)
                    wait64(b)
                    fire_gather(b)
            return carry

        lax.fori_loop(0, nq, quad, 0)
        for c in range(_NB * nq, per):
            b = c % _NB
            wait64(b)
            fire_write(c, b)
        for b in range(_NB):
            wait64(b)

    return mp8, sc_fn


def kernel(x0, x1, b_idxes, i_idxes, j_idxes):
    w0 = W_SIZE
    e = RIGHT_EXTRA
    w1 = w0 + 2 * e
    p0 = PADDING
    p1 = PADDING + e
    N, C, H, W = x0.shape
    ow = (W + 2 * p0 - w0) // STRIDE + 1
    M = b_idxes.shape[0]

    zr = N * H * W
    mp8, sc_fn0 = _build_sc_gather(M, C, H, W, ow, w0, p0, zr)
    _, sc_fn1 = _build_sc_gather(M, C, H, W, ow, w1, p1, zr)
    pad = mp8 - M
    b = jnp.pad(b_idxes.astype(jnp.int32), (0, pad))
    ii = jnp.pad(i_idxes.astype(jnp.int32), (0, pad))
    jj = jnp.pad(j_idxes.astype(jnp.int32), (0, pad))

    t1 = _transpose_to_rows(x1)
    out1f = sc_fn1(t1, b, jj)
    t0 = _transpose_to_rows(x0)
    out0f = sc_fn0(t0, b, ii)
    return (out0f.reshape(M, w0 * w0, C),
            out1f.reshape(M, w1 * w1, C))
